# baseline (device time: 35626 ns/iter reference)
import jax
import jax.numpy as jnp
from jax import lax
from jax.experimental import pallas as pl
from jax.experimental.pallas import tpu as pltpu

N_DEV = 8
N_LAYERS = 3


def kernel(x, Win0, Wout0, Win1, Wout1, Win2, Wout2):
    b, d = x.shape
    rows = b // N_DEV

    def body(x_ref, win0_ref, wout0_ref, win1_ref, wout1_ref, win2_ref,
             wout2_ref, out_ref, send_buf, rs_comm, ag_buf, ag_comm,
             rs_send_sems, rs_recv_sems, ag_send_sems, ag_recv_sems):
        my = lax.axis_index("i")
        wins = [win0_ref, win1_ref, win2_ref]
        wouts = [wout0_ref, wout1_ref, wout2_ref]

        xv = x_ref[...].astype(jnp.bfloat16)
        wb_in = wins[0][...].astype(jnp.bfloat16)
        wb_out = wouts[0][...].astype(jnp.bfloat16)

        for r in range(N_LAYERS):
            h = jnp.dot(xv, wb_in, preferred_element_type=jnp.float32)
            h = jnp.maximum(h, 0.0).astype(jnp.bfloat16)
            partial = jnp.dot(h, wb_out, preferred_element_type=jnp.float32)

            send_buf[r] = partial.astype(jnp.bfloat16).reshape(N_DEV, rows, d)

            for k in range(1, N_DEV):
                tgt = lax.rem(my + k, N_DEV)
                pltpu.make_async_remote_copy(
                    src_ref=send_buf.at[r, pl.ds(tgt, 1)],
                    dst_ref=rs_comm.at[r, pl.ds(my, 1)],
                    send_sem=rs_send_sems.at[r, k - 1],
                    recv_sem=rs_recv_sems.at[r, my],
                    device_id=(tgt,),
                    device_id_type=pl.DeviceIdType.MESH,
                ).start()
            rs_comm[r, pl.ds(my, 1)] = send_buf[r, pl.ds(my, 1)]

            if r < N_LAYERS - 1:
                wb_in = wins[r + 1][...].astype(jnp.bfloat16)
                wb_out = wouts[r + 1][...].astype(jnp.bfloat16)

            for k in range(1, N_DEV):
                src = lax.rem(my + k, N_DEV)
                pltpu.make_async_remote_copy(
                    src_ref=rs_comm.at[r, pl.ds(src, 1)],
                    dst_ref=rs_comm.at[r, pl.ds(src, 1)],
                    send_sem=rs_send_sems.at[r, k - 1],
                    recv_sem=rs_recv_sems.at[r, src],
                    device_id=(src,),
                    device_id_type=pl.DeviceIdType.MESH,
                ).wait_recv()

            red = jnp.sum(rs_comm[r].astype(jnp.float32), axis=0)

            if r < N_LAYERS - 1:
                ag_buf[r] = red.astype(jnp.bfloat16)[None]
                for k in range(1, N_DEV):
                    tgt = lax.rem(my + k, N_DEV)
                    pltpu.make_async_remote_copy(
                        src_ref=ag_buf.at[r],
                        dst_ref=ag_comm.at[r, pl.ds(my, 1)],
                        send_sem=ag_send_sems.at[r, k - 1],
                        recv_sem=ag_recv_sems.at[r, my],
                        device_id=(tgt,),
                        device_id_type=pl.DeviceIdType.MESH,
                    ).start()
                ag_comm[r, pl.ds(my, 1)] = ag_buf[r]
                for k in range(1, N_DEV):
                    src = lax.rem(my + k, N_DEV)
                    pltpu.make_async_remote_copy(
                        src_ref=ag_comm.at[r, pl.ds(src, 1)],
                        dst_ref=ag_comm.at[r, pl.ds(src, 1)],
                        send_sem=ag_send_sems.at[r, k - 1],
                        recv_sem=ag_recv_sems.at[r, src],
                        device_id=(src,),
                        device_id_type=pl.DeviceIdType.MESH,
                    ).wait_recv()
                xv = ag_comm[r].reshape(b, d)
            else:
                out_ref[...] = red

            for k in range(1, N_DEV):
                tgt = lax.rem(my + k, N_DEV)
                pltpu.make_async_remote_copy(
                    src_ref=send_buf.at[r, pl.ds(tgt, 1)],
                    dst_ref=rs_comm.at[r, pl.ds(my, 1)],
                    send_sem=rs_send_sems.at[r, k - 1],
                    recv_sem=rs_recv_sems.at[r, my],
                    device_id=(tgt,),
                    device_id_type=pl.DeviceIdType.MESH,
                ).wait_send()
                if r < N_LAYERS - 1:
                    pltpu.make_async_remote_copy(
                        src_ref=ag_buf.at[r],
                        dst_ref=ag_comm.at[r, pl.ds(my, 1)],
                        send_sem=ag_send_sems.at[r, k - 1],
                        recv_sem=ag_recv_sems.at[r, my],
                        device_id=(tgt,),
                        device_id_type=pl.DeviceIdType.MESH,
                    ).wait_send()

    return pl.pallas_call(
        body,
        out_shape=jax.ShapeDtypeStruct((rows, d), jnp.float32),
        in_specs=[pl.BlockSpec(memory_space=pltpu.VMEM)] * 7,
        out_specs=pl.BlockSpec(memory_space=pltpu.VMEM),
        scratch_shapes=[
            pltpu.VMEM((N_LAYERS, N_DEV, rows, d), jnp.bfloat16),
            pltpu.VMEM((N_LAYERS, N_DEV, rows, d), jnp.bfloat16),
            pltpu.VMEM((N_LAYERS - 1, 1, rows, d), jnp.bfloat16),
            pltpu.VMEM((N_LAYERS - 1, N_DEV, rows, d), jnp.bfloat16),
            pltpu.SemaphoreType.DMA((N_LAYERS, N_DEV - 1)),
            pltpu.SemaphoreType.DMA((N_LAYERS, N_DEV)),
            pltpu.SemaphoreType.DMA((N_LAYERS - 1, N_DEV - 1)),
            pltpu.SemaphoreType.DMA((N_LAYERS - 1, N_DEV)),
        ],
    )(x, Win0, Wout0, Win1, Wout1, Win2, Wout2)


# device time: 32269 ns/iter; 1.1040x vs baseline; 1.1040x over previous
import jax
import jax.numpy as jnp
from jax import lax
from jax.experimental import pallas as pl
from jax.experimental.pallas import tpu as pltpu

N_DEV = 8
N_LAYERS = 3


def kernel(x, Win0, Wout0, Win1, Wout1, Win2, Wout2):
    b, d = x.shape
    rows = b // N_DEV
    hsh = Win0.shape[1]

    def body(x_ref, win0_ref, wout0_ref, win1_ref, wout1_ref, win2_ref,
             wout2_ref, out_ref, wv, ov, send_buf, rs_comm, ag_buf, ag_comm,
             wdma_sems, rs_send_sems, rs_recv_sems, ag_send_sems,
             ag_recv_sems):
        my = lax.axis_index("i")
        wins = [win0_ref, win1_ref, win2_ref]
        wouts = [wout0_ref, wout1_ref, wout2_ref]

        for r in range(N_LAYERS):
            pltpu.make_async_copy(wins[r], wv.at[r], wdma_sems.at[r, 0]).start()
            pltpu.make_async_copy(wouts[r], ov.at[r], wdma_sems.at[r, 1]).start()

        barrier_sem = pltpu.get_barrier_semaphore()
        for k in range(1, N_DEV):
            pl.semaphore_signal(
                barrier_sem, inc=1,
                device_id=(lax.rem(my + k, N_DEV),),
                device_id_type=pl.DeviceIdType.MESH,
            )

        xv = x_ref[...]
        for r in range(N_LAYERS):
            pltpu.make_async_copy(wins[r], wv.at[r], wdma_sems.at[r, 0]).wait()
            h = jnp.dot(xv, wv[r], preferred_element_type=jnp.float32)
            h = jnp.maximum(h, 0.0)
            pltpu.make_async_copy(wouts[r], ov.at[r], wdma_sems.at[r, 1]).wait()
            partial = jnp.dot(h, ov[r], preferred_element_type=jnp.float32)

            send_buf[r] = partial.astype(jnp.bfloat16).reshape(N_DEV, rows, d)

            if r == 0:
                pl.semaphore_wait(barrier_sem, N_DEV - 1)

            for k in range(1, N_DEV):
                tgt = lax.rem(my + k, N_DEV)
                pltpu.make_async_remote_copy(
                    src_ref=send_buf.at[r, pl.ds(tgt, 1)],
                    dst_ref=rs_comm.at[r, pl.ds(my, 1)],
                    send_sem=rs_send_sems.at[r, k - 1],
                    recv_sem=rs_recv_sems.at[r, my],
                    device_id=(tgt,),
                    device_id_type=pl.DeviceIdType.MESH,
                ).start()
            rs_comm[r, pl.ds(my, 1)] = send_buf[r, pl.ds(my, 1)]

            for k in range(1, N_DEV):
                src = lax.rem(my + k, N_DEV)
                pltpu.make_async_remote_copy(
                    src_ref=rs_comm.at[r, pl.ds(src, 1)],
                    dst_ref=rs_comm.at[r, pl.ds(src, 1)],
                    send_sem=rs_send_sems.at[r, k - 1],
                    recv_sem=rs_recv_sems.at[r, src],
                    device_id=(src,),
                    device_id_type=pl.DeviceIdType.MESH,
                ).wait_recv()

            red = jnp.sum(rs_comm[r].astype(jnp.float32), axis=0)

            if r < N_LAYERS - 1:
                ag_buf[r] = red.astype(jnp.bfloat16)[None]
                for k in range(1, N_DEV):
                    tgt = lax.rem(my + k, N_DEV)
                    pltpu.make_async_remote_copy(
                        src_ref=ag_buf.at[r],
                        dst_ref=ag_comm.at[r, pl.ds(my, 1)],
                        send_sem=ag_send_sems.at[r, k - 1],
                        recv_sem=ag_recv_sems.at[r, my],
                        device_id=(tgt,),
                        device_id_type=pl.DeviceIdType.MESH,
                    ).start()
                ag_comm[r, pl.ds(my, 1)] = ag_buf[r]
                for k in range(1, N_DEV):
                    src = lax.rem(my + k, N_DEV)
                    pltpu.make_async_remote_copy(
                        src_ref=ag_comm.at[r, pl.ds(src, 1)],
                        dst_ref=ag_comm.at[r, pl.ds(src, 1)],
                        send_sem=ag_send_sems.at[r, k - 1],
                        recv_sem=ag_recv_sems.at[r, src],
                        device_id=(src,),
                        device_id_type=pl.DeviceIdType.MESH,
                    ).wait_recv()
                xv = ag_comm[r].reshape(b, d).astype(jnp.float32)
            else:
                out_ref[...] = red

            for k in range(1, N_DEV):
                tgt = lax.rem(my + k, N_DEV)
                pltpu.make_async_remote_copy(
                    src_ref=send_buf.at[r, pl.ds(tgt, 1)],
                    dst_ref=rs_comm.at[r, pl.ds(my, 1)],
                    send_sem=rs_send_sems.at[r, k - 1],
                    recv_sem=rs_recv_sems.at[r, my],
                    device_id=(tgt,),
                    device_id_type=pl.DeviceIdType.MESH,
                ).wait_send()
                if r < N_LAYERS - 1:
                    pltpu.make_async_remote_copy(
                        src_ref=ag_buf.at[r],
                        dst_ref=ag_comm.at[r, pl.ds(my, 1)],
                        send_sem=ag_send_sems.at[r, k - 1],
                        recv_sem=ag_recv_sems.at[r, my],
                        device_id=(tgt,),
                        device_id_type=pl.DeviceIdType.MESH,
                    ).wait_send()

    return pl.pallas_call(
        body,
        out_shape=jax.ShapeDtypeStruct((rows, d), jnp.float32),
        in_specs=[pl.BlockSpec(memory_space=pltpu.MemorySpace.VMEM)]
        + [pl.BlockSpec(memory_space=pltpu.MemorySpace.HBM)] * 6,
        out_specs=pl.BlockSpec(memory_space=pltpu.MemorySpace.VMEM),
        scratch_shapes=[
            pltpu.VMEM((N_LAYERS, d, hsh), jnp.float32),
            pltpu.VMEM((N_LAYERS, hsh, d), jnp.float32),
            pltpu.VMEM((N_LAYERS, N_DEV, rows, d), jnp.bfloat16),
            pltpu.VMEM((N_LAYERS, N_DEV, rows, d), jnp.bfloat16),
            pltpu.VMEM((N_LAYERS - 1, 1, rows, d), jnp.bfloat16),
            pltpu.VMEM((N_LAYERS - 1, N_DEV, rows, d), jnp.bfloat16),
            pltpu.SemaphoreType.DMA((N_LAYERS, 2)),
            pltpu.SemaphoreType.DMA((N_LAYERS, N_DEV - 1)),
            pltpu.SemaphoreType.DMA((N_LAYERS, N_DEV)),
            pltpu.SemaphoreType.DMA((N_LAYERS - 1, N_DEV - 1)),
            pltpu.SemaphoreType.DMA((N_LAYERS - 1, N_DEV)),
        ],
        compiler_params=pltpu.CompilerParams(collective_id=0),
    )(x, Win0, Wout0, Win1, Wout1, Win2, Wout2)


# device time: 31829 ns/iter; 1.1193x vs baseline; 1.0138x over previous
import jax
import jax.numpy as jnp
from jax import lax
from jax.experimental import pallas as pl
from jax.experimental.pallas import tpu as pltpu

N_DEV = 8
N_LAYERS = 3
HALF = N_DEV // 2


def kernel(x, Win0, Wout0, Win1, Wout1, Win2, Wout2):
    b, d = x.shape
    rows = b // N_DEV
    hsh = Win0.shape[1]

    def body(x_ref, win0_ref, wout0_ref, win1_ref, wout1_ref, win2_ref,
             wout2_ref, out_ref, wv, ov, send_buf, rs_comm, ag_buf, ag_comm,
             wdma_sems, rs_send_sems, rs_recv_sems, ag_send_sems,
             ag_recv_sems):
        my = lax.axis_index("i")
        wins = [win0_ref, win1_ref, win2_ref]
        wouts = [wout0_ref, wout1_ref, wout2_ref]

        for r in range(N_LAYERS):
            pltpu.make_async_copy(wins[r], wv.at[r], wdma_sems.at[r, 0]).start()
            pltpu.make_async_copy(wouts[r], ov.at[r], wdma_sems.at[r, 1]).start()

        barrier_sem = pltpu.get_barrier_semaphore()
        for s in range(N_DEV):
            @pl.when(my != s)
            def _():
                pl.semaphore_signal(
                    barrier_sem, inc=1, device_id=(s,),
                    device_id_type=pl.DeviceIdType.MESH,
                )

        def rs_issue(r, lo, hi):
            for s in range(lo, hi):
                @pl.when(my != s)
                def _():
                    pltpu.make_async_remote_copy(
                        src_ref=send_buf.at[r, pl.ds(s, 1)],
                        dst_ref=rs_comm.at[r, pl.ds(my, 1)],
                        send_sem=rs_send_sems.at[r, s],
                        recv_sem=rs_recv_sems.at[r, my],
                        device_id=(s,),
                        device_id_type=pl.DeviceIdType.MESH,
                    ).start()

        def rs_finish(r):
            rs_comm[r, pl.ds(my, 1)] = send_buf[r, pl.ds(my, 1)]
            for s in range(N_DEV):
                @pl.when(my != s)
                def _():
                    pltpu.make_async_remote_copy(
                        src_ref=rs_comm.at[r, pl.ds(s, 1)],
                        dst_ref=rs_comm.at[r, pl.ds(s, 1)],
                        send_sem=rs_send_sems.at[r, s],
                        recv_sem=rs_recv_sems.at[r, s],
                        device_id=(s,),
                        device_id_type=pl.DeviceIdType.MESH,
                    ).wait_recv()
            return jnp.sum(rs_comm[r].astype(jnp.float32), axis=0)

        def ag_issue(r, red):
            ag_buf[r] = red.astype(jnp.bfloat16)[None]
            for s in range(N_DEV):
                @pl.when(my != s)
                def _():
                    pltpu.make_async_remote_copy(
                        src_ref=ag_buf.at[r],
                        dst_ref=ag_comm.at[r, pl.ds(my, 1)],
                        send_sem=ag_send_sems.at[r, s],
                        recv_sem=ag_recv_sems.at[r, my],
                        device_id=(s,),
                        device_id_type=pl.DeviceIdType.MESH,
                    ).start()
            ag_comm[r, pl.ds(my, 1)] = ag_buf[r]

        def ag_wait(r, lo, hi):
            for s in range(lo, hi):
                @pl.when(my != s)
                def _():
                    pltpu.make_async_remote_copy(
                        src_ref=ag_comm.at[r, pl.ds(s, 1)],
                        dst_ref=ag_comm.at[r, pl.ds(s, 1)],
                        send_sem=ag_send_sems.at[r, s],
                        recv_sem=ag_recv_sems.at[r, s],
                        device_id=(s,),
                        device_id_type=pl.DeviceIdType.MESH,
                    ).wait_recv()

        def layer_half(r, xh):
            h = jnp.dot(xh, wv[r], preferred_element_type=jnp.float32)
            h = jnp.maximum(h, 0.0)
            return jnp.dot(h, ov[r], preferred_element_type=jnp.float32)

        xv = x_ref[...]
        pltpu.make_async_copy(wins[0], wv.at[0], wdma_sems.at[0, 0]).wait()
        h = jnp.maximum(jnp.dot(xv, wv[0], preferred_element_type=jnp.float32), 0.0)
        pltpu.make_async_copy(wouts[0], ov.at[0], wdma_sems.at[0, 1]).wait()
        partial = jnp.dot(h, ov[0], preferred_element_type=jnp.float32)
        send_buf[0] = partial.astype(jnp.bfloat16).reshape(N_DEV, rows, d)

        pl.semaphore_wait(barrier_sem, N_DEV - 1)
        rs_issue(0, 0, N_DEV)
        red = rs_finish(0)
        ag_issue(0, red)

        for r in range(1, N_LAYERS):
            pltpu.make_async_copy(wins[r], wv.at[r], wdma_sems.at[r, 0]).wait()
            pltpu.make_async_copy(wouts[r], ov.at[r], wdma_sems.at[r, 1]).wait()
            for half in range(2):
                lo, hi = half * HALF, (half + 1) * HALF
                ag_wait(r - 1, lo, hi)
                xh = ag_comm[r - 1, lo:hi].reshape(HALF * rows, d)
                ph = layer_half(r, xh.astype(jnp.float32))
                send_buf[r, lo:hi] = ph.astype(jnp.bfloat16).reshape(
                    HALF, rows, d)
                rs_issue(r, lo, hi)
            red = rs_finish(r)
            if r < N_LAYERS - 1:
                ag_issue(r, red)
            else:
                out_ref[...] = red

        for r in range(N_LAYERS):
            for s in range(N_DEV):
                @pl.when(my != s)
                def _():
                    pltpu.make_async_remote_copy(
                        src_ref=send_buf.at[r, pl.ds(s, 1)],
                        dst_ref=rs_comm.at[r, pl.ds(my, 1)],
                        send_sem=rs_send_sems.at[r, s],
                        recv_sem=rs_recv_sems.at[r, my],
                        device_id=(s,),
                        device_id_type=pl.DeviceIdType.MESH,
                    ).wait_send()
                if r < N_LAYERS - 1:
                    @pl.when(my != s)
                    def _():
                        pltpu.make_async_remote_copy(
                            src_ref=ag_buf.at[r],
                            dst_ref=ag_comm.at[r, pl.ds(my, 1)],
                            send_sem=ag_send_sems.at[r, s],
                            recv_sem=ag_recv_sems.at[r, my],
                            device_id=(s,),
                            device_id_type=pl.DeviceIdType.MESH,
                        ).wait_send()

    return pl.pallas_call(
        body,
        out_shape=jax.ShapeDtypeStruct((rows, d), jnp.float32),
        in_specs=[pl.BlockSpec(memory_space=pltpu.MemorySpace.VMEM)]
        + [pl.BlockSpec(memory_space=pltpu.MemorySpace.HBM)] * 6,
        out_specs=pl.BlockSpec(memory_space=pltpu.MemorySpace.VMEM),
        scratch_shapes=[
            pltpu.VMEM((N_LAYERS, d, hsh), jnp.float32),
            pltpu.VMEM((N_LAYERS, hsh, d), jnp.float32),
            pltpu.VMEM((N_LAYERS, N_DEV, rows, d), jnp.bfloat16),
            pltpu.VMEM((N_LAYERS, N_DEV, rows, d), jnp.bfloat16),
            pltpu.VMEM((N_LAYERS - 1, 1, rows, d), jnp.bfloat16),
            pltpu.VMEM((N_LAYERS - 1, N_DEV, rows, d), jnp.bfloat16),
            pltpu.SemaphoreType.DMA((N_LAYERS, 2)),
            pltpu.SemaphoreType.DMA((N_LAYERS, N_DEV)),
            pltpu.SemaphoreType.DMA((N_LAYERS, N_DEV)),
            pltpu.SemaphoreType.DMA((N_LAYERS, N_DEV)),
            pltpu.SemaphoreType.DMA((N_LAYERS, N_DEV)),
        ],
        compiler_params=pltpu.CompilerParams(collective_id=0),
    )(x, Win0, Wout0, Win1, Wout1, Win2, Wout2)


# device time: 24838 ns/iter; 1.4343x vs baseline; 1.2815x over previous
import jax
import jax.numpy as jnp
from jax import lax
from jax.experimental import pallas as pl
from jax.experimental.pallas import tpu as pltpu

N_DEV = 8
N_LAYERS = 3
HALF = N_DEV // 2


def kernel(x, Win0, Wout0, Win1, Wout1, Win2, Wout2):
    b, d = x.shape
    rows = b // N_DEV
    hsh = Win0.shape[1]

    def body(x_ref, win0_ref, wout0_ref, win1_ref, wout1_ref, win2_ref,
             wout2_ref, out_ref, wv, ov, send_buf, rs_comm, ag_buf, ag_comm,
             xv_ref, wdma_sems, xdma_sem, rs_send_sems, rs_recv_sems,
             ag_send_sems, ag_recv_sems):
        my = lax.axis_index("i")
        wins = [win0_ref, win1_ref, win2_ref]
        wouts = [wout0_ref, wout1_ref, wout2_ref]

        pltpu.make_async_copy(x_ref, xv_ref, xdma_sem).start()
        for r in range(N_LAYERS):
            pltpu.make_async_copy(wins[r], wv.at[r], wdma_sems.at[r, 0]).start()
            pltpu.make_async_copy(wouts[r], ov.at[r], wdma_sems.at[r, 1]).start()

        barrier_sem = pltpu.get_barrier_semaphore()
        for s in range(N_DEV):
            @pl.when(my != s)
            def _():
                pl.semaphore_signal(
                    barrier_sem, inc=1, device_id=(s,),
                    device_id_type=pl.DeviceIdType.MESH,
                )

        def rs_issue(r, lo, hi):
            for s in range(lo, hi):
                @pl.when(my != s)
                def _():
                    pltpu.make_async_remote_copy(
                        src_ref=send_buf.at[r, pl.ds(s, 1)],
                        dst_ref=rs_comm.at[r, pl.ds(my, 1)],
                        send_sem=rs_send_sems.at[r, s],
                        recv_sem=rs_recv_sems.at[r, my],
                        device_id=(s,),
                        device_id_type=pl.DeviceIdType.MESH,
                    ).start()

        def rs_finish(r):
            rs_comm[r, pl.ds(my, 1)] = send_buf[r, pl.ds(my, 1)]
            for s in range(N_DEV):
                @pl.when(my != s)
                def _():
                    pltpu.make_async_remote_copy(
                        src_ref=rs_comm.at[r, pl.ds(s, 1)],
                        dst_ref=rs_comm.at[r, pl.ds(s, 1)],
                        send_sem=rs_send_sems.at[r, s],
                        recv_sem=rs_recv_sems.at[r, s],
                        device_id=(s,),
                        device_id_type=pl.DeviceIdType.MESH,
                    ).wait_recv()
            return jnp.sum(rs_comm[r].astype(jnp.float32), axis=0)

        def ag_issue(r, red):
            ag_buf[r] = red.astype(jnp.bfloat16)[None]
            for s in range(N_DEV):
                @pl.when(my != s)
                def _():
                    pltpu.make_async_remote_copy(
                        src_ref=ag_buf.at[r],
                        dst_ref=ag_comm.at[r, pl.ds(my, 1)],
                        send_sem=ag_send_sems.at[r, s],
                        recv_sem=ag_recv_sems.at[r, my],
                        device_id=(s,),
                        device_id_type=pl.DeviceIdType.MESH,
                    ).start()
            ag_comm[r, pl.ds(my, 1)] = ag_buf[r]

        def ag_wait(r, lo, hi):
            for s in range(lo, hi):
                @pl.when(my != s)
                def _():
                    pltpu.make_async_remote_copy(
                        src_ref=ag_comm.at[r, pl.ds(s, 1)],
                        dst_ref=ag_comm.at[r, pl.ds(s, 1)],
                        send_sem=ag_send_sems.at[r, s],
                        recv_sem=ag_recv_sems.at[r, s],
                        device_id=(s,),
                        device_id_type=pl.DeviceIdType.MESH,
                    ).wait_recv()

        def layer_half(r, xh):
            h = jnp.dot(xh, wv[r], preferred_element_type=jnp.float32)
            h = jnp.maximum(h, 0.0)
            return jnp.dot(h, ov[r], preferred_element_type=jnp.float32)

        pltpu.make_async_copy(x_ref, xv_ref, xdma_sem).wait()
        xv = xv_ref[...]
        pltpu.make_async_copy(wins[0], wv.at[0], wdma_sems.at[0, 0]).wait()
        h = jnp.maximum(jnp.dot(xv, wv[0], preferred_element_type=jnp.float32), 0.0)
        pltpu.make_async_copy(wouts[0], ov.at[0], wdma_sems.at[0, 1]).wait()
        partial = jnp.dot(h, ov[0], preferred_element_type=jnp.float32)
        send_buf[0] = partial.astype(jnp.bfloat16).reshape(N_DEV, rows, d)

        pl.semaphore_wait(barrier_sem, N_DEV - 1)
        rs_issue(0, 0, N_DEV)
        red = rs_finish(0)
        ag_issue(0, red)

        for r in range(1, N_LAYERS):
            pltpu.make_async_copy(wins[r], wv.at[r], wdma_sems.at[r, 0]).wait()
            pltpu.make_async_copy(wouts[r], ov.at[r], wdma_sems.at[r, 1]).wait()
            for half in range(2):
                lo, hi = half * HALF, (half + 1) * HALF
                ag_wait(r - 1, lo, hi)
                xh = ag_comm[r - 1, lo:hi].reshape(HALF * rows, d)
                ph = layer_half(r, xh.astype(jnp.float32))
                send_buf[r, lo:hi] = ph.astype(jnp.bfloat16).reshape(
                    HALF, rows, d)
                rs_issue(r, lo, hi)
            red = rs_finish(r)
            if r < N_LAYERS - 1:
                ag_issue(r, red)
            else:
                out_ref[...] = red

        for r in range(N_LAYERS):
            for s in range(N_DEV):
                @pl.when(my != s)
                def _():
                    pltpu.make_async_remote_copy(
                        src_ref=send_buf.at[r, pl.ds(s, 1)],
                        dst_ref=rs_comm.at[r, pl.ds(my, 1)],
                        send_sem=rs_send_sems.at[r, s],
                        recv_sem=rs_recv_sems.at[r, my],
                        device_id=(s,),
                        device_id_type=pl.DeviceIdType.MESH,
                    ).wait_send()
                if r < N_LAYERS - 1:
                    @pl.when(my != s)
                    def _():
                        pltpu.make_async_remote_copy(
                            src_ref=ag_buf.at[r],
                            dst_ref=ag_comm.at[r, pl.ds(my, 1)],
                            send_sem=ag_send_sems.at[r, s],
                            recv_sem=ag_recv_sems.at[r, my],
                            device_id=(s,),
                            device_id_type=pl.DeviceIdType.MESH,
                        ).wait_send()

    return pl.pallas_call(
        body,
        out_shape=jax.ShapeDtypeStruct((rows, d), jnp.float32),
        in_specs=[pl.BlockSpec(memory_space=pltpu.MemorySpace.HBM)] * 7,
        out_specs=pl.BlockSpec(memory_space=pltpu.MemorySpace.VMEM),
        scratch_shapes=[
            pltpu.VMEM((N_LAYERS, d, hsh), jnp.float32),
            pltpu.VMEM((N_LAYERS, hsh, d), jnp.float32),
            pltpu.VMEM((N_LAYERS, N_DEV, rows, d), jnp.bfloat16),
            pltpu.VMEM((N_LAYERS, N_DEV, rows, d), jnp.bfloat16),
            pltpu.VMEM((N_LAYERS - 1, 1, rows, d), jnp.bfloat16),
            pltpu.VMEM((N_LAYERS - 1, N_DEV, rows, d), jnp.bfloat16),
            pltpu.VMEM((b, d), jnp.float32),
            pltpu.SemaphoreType.DMA((N_LAYERS, 2)),
            pltpu.SemaphoreType.DMA(()),
            pltpu.SemaphoreType.DMA((N_LAYERS, N_DEV)),
            pltpu.SemaphoreType.DMA((N_LAYERS, N_DEV)),
            pltpu.SemaphoreType.DMA((N_LAYERS, N_DEV)),
            pltpu.SemaphoreType.DMA((N_LAYERS, N_DEV)),
        ],
        compiler_params=pltpu.CompilerParams(collective_id=0),
    )(*(pltpu.with_memory_space_constraint(a, pltpu.MemorySpace.HBM)
        for a in (x, Win0, Wout0, Win1, Wout1, Win2, Wout2)))


# device time: 22700 ns/iter; 1.5694x vs baseline; 1.0942x over previous
import jax
import jax.numpy as jnp
from jax import lax
from jax.experimental import pallas as pl
from jax.experimental.pallas import tpu as pltpu

N_DEV = 8
N_LAYERS = 3
HALF = N_DEV // 2


def kernel(x, Win0, Wout0, Win1, Wout1, Win2, Wout2):
    b, d = x.shape
    rows = b // N_DEV
    hsh = Win0.shape[1]

    def body(x_ref, win0_ref, wout0_ref, win1_ref, wout1_ref, win2_ref,
             wout2_ref, out_ref, wv, ov, send_buf, rs_comm, ag_buf, ag_comm,
             xv_ref, wdma_sems, w0_sems, xdma_sem, rs_send_sems,
             rs_recv_sems, ag_send_sems, ag_recv_sems):
        my = lax.axis_index("i")
        wins = [win0_ref, win1_ref, win2_ref]
        wouts = [wout0_ref, wout1_ref, wout2_ref]

        h2 = hsh // 2

        def w0_copies():
            return [
                pltpu.make_async_copy(wins[0].at[:, pl.ds(0, h2)],
                                      wv.at[0, :, pl.ds(0, h2)],
                                      w0_sems.at[0]),
                pltpu.make_async_copy(wouts[0].at[pl.ds(0, h2)],
                                      ov.at[0, pl.ds(0, h2)],
                                      w0_sems.at[1]),
                pltpu.make_async_copy(wins[0].at[:, pl.ds(h2, h2)],
                                      wv.at[0, :, pl.ds(h2, h2)],
                                      w0_sems.at[2]),
                pltpu.make_async_copy(wouts[0].at[pl.ds(h2, h2)],
                                      ov.at[0, pl.ds(h2, h2)],
                                      w0_sems.at[3]),
            ]

        pltpu.make_async_copy(x_ref, xv_ref, xdma_sem).start()
        for c in w0_copies():
            c.start()

        barrier_sem = pltpu.get_barrier_semaphore()
        for s in range(N_DEV):
            @pl.when(my != s)
            def _():
                pl.semaphore_signal(
                    barrier_sem, inc=1, device_id=(s,),
                    device_id_type=pl.DeviceIdType.MESH,
                )

        def rs_issue(r, lo, hi):
            for s in range(lo, hi):
                @pl.when(my != s)
                def _():
                    pltpu.make_async_remote_copy(
                        src_ref=send_buf.at[r, pl.ds(s, 1)],
                        dst_ref=rs_comm.at[r, pl.ds(my, 1)],
                        send_sem=rs_send_sems.at[r, s],
                        recv_sem=rs_recv_sems.at[r, my],
                        device_id=(s,),
                        device_id_type=pl.DeviceIdType.MESH,
                    ).start()

        def rs_finish(r):
            rs_comm[r, pl.ds(my, 1)] = send_buf[r, pl.ds(my, 1)]
            for s in range(N_DEV):
                @pl.when(my != s)
                def _():
                    pltpu.make_async_remote_copy(
                        src_ref=rs_comm.at[r, pl.ds(s, 1)],
                        dst_ref=rs_comm.at[r, pl.ds(s, 1)],
                        send_sem=rs_send_sems.at[r, s],
                        recv_sem=rs_recv_sems.at[r, s],
                        device_id=(s,),
                        device_id_type=pl.DeviceIdType.MESH,
                    ).wait_recv()
            return jnp.sum(rs_comm[r].astype(jnp.float32), axis=0)

        def ag_issue(r, red):
            ag_buf[r] = red.astype(jnp.bfloat16)[None]
            for s in range(N_DEV):
                @pl.when(my != s)
                def _():
                    pltpu.make_async_remote_copy(
                        src_ref=ag_buf.at[r],
                        dst_ref=ag_comm.at[r, pl.ds(my, 1)],
                        send_sem=ag_send_sems.at[r, s],
                        recv_sem=ag_recv_sems.at[r, my],
                        device_id=(s,),
                        device_id_type=pl.DeviceIdType.MESH,
                    ).start()
            ag_comm[r, pl.ds(my, 1)] = ag_buf[r]

        def ag_wait(r, lo, hi):
            for s in range(lo, hi):
                @pl.when(my != s)
                def _():
                    pltpu.make_async_remote_copy(
                        src_ref=ag_comm.at[r, pl.ds(s, 1)],
                        dst_ref=ag_comm.at[r, pl.ds(s, 1)],
                        send_sem=ag_send_sems.at[r, s],
                        recv_sem=ag_recv_sems.at[r, s],
                        device_id=(s,),
                        device_id_type=pl.DeviceIdType.MESH,
                    ).wait_recv()

        def layer_half(r, xh):
            h = jnp.dot(xh, wv[r], preferred_element_type=jnp.float32)
            h = jnp.maximum(h, 0.0)
            return jnp.dot(h, ov[r], preferred_element_type=jnp.float32)

        c0, c1, c2, c3 = w0_copies()
        pltpu.make_async_copy(x_ref, xv_ref, xdma_sem).wait()
        xv = xv_ref[...]
        c0.wait()
        ha = jnp.maximum(jnp.dot(xv, wv[0, :, :h2],
                                 preferred_element_type=jnp.float32), 0.0)
        c1.wait()
        partial = jnp.dot(ha, ov[0, :h2], preferred_element_type=jnp.float32)
        c2.wait()
        hb = jnp.maximum(jnp.dot(xv, wv[0, :, h2:],
                                 preferred_element_type=jnp.float32), 0.0)
        c3.wait()
        partial = partial + jnp.dot(hb, ov[0, h2:],
                                    preferred_element_type=jnp.float32)
        send_buf[0] = partial.astype(jnp.bfloat16).reshape(N_DEV, rows, d)

        pl.semaphore_wait(barrier_sem, N_DEV - 1)
        rs_issue(0, 0, N_DEV)
        for r in range(1, N_LAYERS):
            pltpu.make_async_copy(wins[r], wv.at[r], wdma_sems.at[r, 0]).start()
            pltpu.make_async_copy(wouts[r], ov.at[r], wdma_sems.at[r, 1]).start()
        red = rs_finish(0)
        ag_issue(0, red)

        for r in range(1, N_LAYERS):
            pltpu.make_async_copy(wins[r], wv.at[r], wdma_sems.at[r, 0]).wait()
            pltpu.make_async_copy(wouts[r], ov.at[r], wdma_sems.at[r, 1]).wait()
            for half in range(2):
                lo, hi = half * HALF, (half + 1) * HALF
                ag_wait(r - 1, lo, hi)
                xh = ag_comm[r - 1, lo:hi].reshape(HALF * rows, d)
                ph = layer_half(r, xh.astype(jnp.float32))
                send_buf[r, lo:hi] = ph.astype(jnp.bfloat16).reshape(
                    HALF, rows, d)
                rs_issue(r, lo, hi)
            red = rs_finish(r)
            if r < N_LAYERS - 1:
                ag_issue(r, red)
            else:
                out_ref[...] = red

        for r in range(N_LAYERS):
            for s in range(N_DEV):
                @pl.when(my != s)
                def _():
                    pltpu.make_async_remote_copy(
                        src_ref=send_buf.at[r, pl.ds(s, 1)],
                        dst_ref=rs_comm.at[r, pl.ds(my, 1)],
                        send_sem=rs_send_sems.at[r, s],
                        recv_sem=rs_recv_sems.at[r, my],
                        device_id=(s,),
                        device_id_type=pl.DeviceIdType.MESH,
                    ).wait_send()
                if r < N_LAYERS - 1:
                    @pl.when(my != s)
                    def _():
                        pltpu.make_async_remote_copy(
                            src_ref=ag_buf.at[r],
                            dst_ref=ag_comm.at[r, pl.ds(my, 1)],
                            send_sem=ag_send_sems.at[r, s],
                            recv_sem=ag_recv_sems.at[r, my],
                            device_id=(s,),
                            device_id_type=pl.DeviceIdType.MESH,
                        ).wait_send()

    return pl.pallas_call(
        body,
        out_shape=jax.ShapeDtypeStruct((rows, d), jnp.float32),
        in_specs=[pl.BlockSpec(memory_space=pltpu.MemorySpace.HBM)] * 7,
        out_specs=pl.BlockSpec(memory_space=pltpu.MemorySpace.VMEM),
        scratch_shapes=[
            pltpu.VMEM((N_LAYERS, d, hsh), jnp.float32),
            pltpu.VMEM((N_LAYERS, hsh, d), jnp.float32),
            pltpu.VMEM((N_LAYERS, N_DEV, rows, d), jnp.bfloat16),
            pltpu.VMEM((N_LAYERS, N_DEV, rows, d), jnp.bfloat16),
            pltpu.VMEM((N_LAYERS - 1, 1, rows, d), jnp.bfloat16),
            pltpu.VMEM((N_LAYERS - 1, N_DEV, rows, d), jnp.bfloat16),
            pltpu.VMEM((b, d), jnp.float32),
            pltpu.SemaphoreType.DMA((N_LAYERS, 2)),
            pltpu.SemaphoreType.DMA((4,)),
            pltpu.SemaphoreType.DMA(()),
            pltpu.SemaphoreType.DMA((N_LAYERS, N_DEV)),
            pltpu.SemaphoreType.DMA((N_LAYERS, N_DEV)),
            pltpu.SemaphoreType.DMA((N_LAYERS, N_DEV)),
            pltpu.SemaphoreType.DMA((N_LAYERS, N_DEV)),
        ],
        compiler_params=pltpu.CompilerParams(collective_id=0),
    )(*(pltpu.with_memory_space_constraint(a, pltpu.MemorySpace.HBM)
        for a in (x, Win0, Wout0, Win1, Wout1, Win2, Wout2)))
